# baseline (speedup 1.0000x reference)
import functools

import jax
import jax.numpy as jnp
from jax import lax
from jax.experimental import pallas as pl
from jax.experimental.pallas import tpu as pltpu
from jax.experimental.pallas import tpu_sc as plsc

B = 1
N0 = 512
DIM = 128
HID = 256
RC = 8
RS = 4
K = 16

_DOT = functools.partial(
    jax.lax.dot_general, precision=jax.lax.Precision.DEFAULT,
    preferred_element_type=jnp.float32)


def _mm(a, b):
    return _DOT(a, b, (((1,), (0,)), ((), ())))


def _layer_body(xyz_ref, f_ref, w1_ref, b1_ref, wx_ref, bx_ref, wf_ref,
                bf_ref, nw1_ref, nb1_ref, nw2_ref, nb2_ref, rw1_ref, rb1_ref,
                rwf_ref, rbf_ref, rwx_ref, rbx_ref,
                xyz_out_ref, f_out_ref, un_out_ref):
    x = xyz_ref[...]
    f = f_ref[...]
    h = jnp.maximum(_mm(f, w1_ref[...]) + b1_ref[...], 0.0)
    co = _mm(h, wx_ref[...]) + bx_ref[...]
    cf = _mm(h, wf_ref[...]) + bf_ref[...]
    hn = jnp.maximum(_mm(f, nw1_ref[...]) + nb1_ref[...], 0.0)
    logit = _mm(hn, nw2_ref[...]) + nb2_ref[...]
    un = 1.0 + (RC - 1.0) * jax.nn.sigmoid(logit[:, 0:1])
    un_out_ref[...] = un
    for r in range(RS):
        xyz_c = x + co[:, r * 128:(r + 1) * 128]
        m_r = jax.nn.sigmoid(un - (r + 1.0))
        f_c = cf[:, r * 128:(r + 1) * 128] * m_r
        h2 = jnp.maximum(_mm(f_c, rw1_ref[...]) + rb1_ref[...], 0.0)
        f_out_ref[:, r, :] = f_c + _mm(h2, rwf_ref[...]) + rbf_ref[...]
        xyz_out_ref[:, r, :] = xyz_c + _mm(h2, rwx_ref[...]) + rbx_ref[...]


def _run_layer(xyzp, f, w1, b1, wxb, bxb, wf4, bf4, nw1, nb1, nw2p, nb2p,
               rw1, rb1, rwf, rbf, rwxp, rbxp, bn):
    n = f.shape[0]
    grid = (n // bn,)
    row = lambda i: (i, 0)
    row3 = lambda i: (i, 0, 0)
    full2 = lambda i: (0, 0)
    wspec = lambda a: pl.BlockSpec(a.shape, full2)
    in_specs = [
        pl.BlockSpec((bn, 128), row), pl.BlockSpec((bn, 128), row),
        wspec(w1), wspec(b1), wspec(wxb), wspec(bxb), wspec(wf4), wspec(bf4),
        wspec(nw1), wspec(nb1), wspec(nw2p), wspec(nb2p),
        wspec(rw1), wspec(rb1), wspec(rwf), wspec(rbf), wspec(rwxp),
        wspec(rbxp),
    ]
    out_shape = [
        jax.ShapeDtypeStruct((n, RS, 128), jnp.float32),
        jax.ShapeDtypeStruct((n, RS, 128), jnp.float32),
        jax.ShapeDtypeStruct((n, 1), jnp.float32),
    ]
    out_specs = [
        pl.BlockSpec((bn, RS, 128), row3),
        pl.BlockSpec((bn, RS, 128), row3),
        pl.BlockSpec((bn, 1), row),
    ]
    xyz_out, f_out, un = pl.pallas_call(
        _layer_body, grid=grid, in_specs=in_specs, out_specs=out_specs,
        out_shape=out_shape)(
            xyzp, f, w1, b1, wxb, bxb, wf4, bf4, nw1, nb1, nw2p, nb2p,
            rw1, rb1, rwf, rbf, rwxp, rbxp)
    return (xyz_out.reshape(n * RS, 128), f_out.reshape(n * RS, 128), un)


def _knn_body(prev_ref, currt_ref, un_ref, mdis_ref):
    pv = prev_ref[...]
    ct = currt_ref[...]
    pn = jnp.sum(pv * pv, axis=1, keepdims=True)
    cn = jnp.sum(ct * ct, axis=0, keepdims=True)
    d2 = pn + cn - 2.0 * _mm(pv, ct)
    big = jnp.float32(3.0e38)
    acc = jnp.zeros_like(pn)
    rem = jnp.full_like(pn, float(K))
    vals = d2
    for _ in range(K):
        m = jnp.min(vals, axis=1, keepdims=True)
        eq = vals == m
        c = jnp.sum(eq.astype(jnp.float32), axis=1, keepdims=True)
        t = jnp.minimum(c, rem)
        acc = acc + jnp.where(
            t > 0.0, jnp.sqrt(jnp.maximum(m, 1e-12)) * t, 0.0)
        rem = rem - t
        vals = jnp.where(eq, big, vals)
    mdis_ref[...] = acc / un_ref[...]


def _run_knn(prevp, currp, un, bp):
    p = prevp.shape[0]
    c = currp.shape[0]
    currt = currp.T
    grid = (p // bp,)
    row = lambda i: (i, 0)
    full2 = lambda i: (0, 0)
    mdis = pl.pallas_call(
        _knn_body, grid=grid,
        in_specs=[pl.BlockSpec((bp, 128), row),
                  pl.BlockSpec((128, c), full2),
                  pl.BlockSpec((bp, 1), row)],
        out_specs=pl.BlockSpec((bp, 1), row),
        out_shape=jax.ShapeDtypeStruct((p, 1), jnp.float32),
    )(prevp, currt, un)
    return mdis


def _d2_body(prev_ref, currt_ref, out_ref):
    pv = prev_ref[...]
    ct = currt_ref[...]
    pn = jnp.sum(pv * pv, axis=1, keepdims=True)
    cn = jnp.sum(ct * ct, axis=0, keepdims=True)
    out_ref[...] = pn + cn - 2.0 * _mm(pv, ct)


def _run_d2(prevp, currp, bp):
    p = prevp.shape[0]
    c = currp.shape[0]
    currt = currp.T
    row = lambda i: (i, 0)
    full2 = lambda i: (0, 0)
    return pl.pallas_call(
        _d2_body, grid=(p // bp,),
        in_specs=[pl.BlockSpec((bp, 128), row),
                  pl.BlockSpec((128, c), full2)],
        out_specs=pl.BlockSpec((bp, c), row),
        out_shape=jax.ShapeDtypeStruct((p, c), jnp.float32),
    )(prevp, currt)


def _sc_iota():
    return lax.iota(jnp.int32, 16)


def _sc_gather_xor(v, j):
    idx = _sc_iota() ^ j
    return lax.gather(
        v, idx[:, None],
        dimension_numbers=lax.GatherDimensionNumbers(
            offset_dims=(), collapsed_slice_dims=(0,), start_index_map=(0,)),
        slice_sizes=(1,), mode=lax.GatherScatterMode.PROMISE_IN_BOUNDS)


def _sc_bfly(v, op):
    for j in (1, 2, 4, 8):
        v = op(v, _sc_gather_xor(v, j))
    return v


def _sc_sort16(v):
    io = _sc_iota()
    for k in (2, 4, 8, 16):
        lk = k.bit_length() - 1
        j = k // 2
        while j >= 1:
            lj = j.bit_length() - 1
            pr = _sc_gather_xor(v, j)
            lo = jnp.minimum(v, pr)
            hi = jnp.maximum(v, pr)
            dirpos = ((io >> lk) ^ (io >> lj)) & 1
            v = jnp.where(dirpos == 0, lo, hi)
            j //= 2
    return v


def _sc_merge16(best, vs):
    io = _sc_iota()
    b2 = jnp.minimum(best, lax.rev(vs, (0,)))
    for j in (8, 4, 2, 1):
        pr = _sc_gather_xor(b2, j)
        lo = jnp.minimum(b2, pr)
        hi = jnp.maximum(b2, pr)
        b2 = jnp.where((io & j) == 0, lo, hi)
    return b2


_SC_BIG = 3.0e38


def _run_knn_sc(d2, rows_hint=None):
    p_num, c_num = d2.shape
    nw = 32
    rpw = p_num // nw
    nsup = c_num // 128
    d2flat = d2.reshape(-1)
    mesh = plsc.VectorSubcoreMesh(core_axis_name="c", subcore_axis_name="s")

    @functools.partial(
        pl.kernel, mesh=mesh,
        out_type=jax.ShapeDtypeStruct((p_num * 16,), jnp.float32),
        scratch_types=[
            pltpu.VMEM((c_num,), jnp.float32),
            pltpu.VMEM((c_num,), jnp.float32),
            pltpu.VMEM((c_num,), jnp.float32),
            pltpu.VMEM((rpw * 16,), jnp.float32),
            pltpu.SMEM((2,), jnp.int32),
        ])
    def knn(d2_hbm, out_hbm, rowv, buf, buf2, bestv, sp):
        wid = lax.axis_index("s") * 2 + lax.axis_index("c")
        base = wid * rpw

        def row_body(i, carry0):
            pltpu.sync_copy(d2_hbm.at[pl.ds((base + i) * c_num, c_num)],
                            rowv)
            sp[0] = 0

            def sup_body(s_, rm):
                off0 = s_ * 128
                d = [rowv[pl.ds(off0 + j * 16, 16)] for j in range(8)]
                m01 = jnp.minimum(d[0], d[1])
                m23 = jnp.minimum(d[2], d[3])
                m45 = jnp.minimum(d[4], d[5])
                m67 = jnp.minimum(d[6], d[7])
                runm = jnp.minimum(jnp.minimum(m01, m23),
                                   jnp.minimum(m45, m67))
                rm2 = jnp.minimum(rm, runm)
                t_now = _sc_bfly(rm2, jnp.maximum)[0]
                smin = _sc_bfly(runm, jnp.minimum)[0]

                @pl.when(smin <= t_now)
                def _():
                    ptr = sp[0]
                    for j in range(8):
                        buf[pl.ds(ptr + j * 16, 16)] = d[j]
                    sp[0] = ptr + 128

                return rm2

            rm_fin = lax.fori_loop(
                0, nsup, sup_body, jnp.full((16,), _SC_BIG, jnp.float32))
            t_fin = _sc_bfly(rm_fin, jnp.maximum)[0]
            nb = sp[0]
            sp[1] = 0

            def refil_body(j, carry):
                v = buf[pl.ds(j * 16, 16)]
                mn = _sc_bfly(v, jnp.minimum)[0]

                @pl.when(mn <= t_fin)
                def _():
                    q = sp[1]
                    buf2[pl.ds(q, 16)] = v
                    sp[1] = q + 16

                return carry

            lax.fori_loop(0, nb // 16, refil_body, 0)

            def merge_body(j, best):
                vs = _sc_sort16(buf2[pl.ds(j * 16, 16)])
                return _sc_merge16(best, vs)

            best = lax.fori_loop(0, sp[1] // 16, merge_body,
                                 jnp.full((16,), _SC_BIG, jnp.float32))
            bestv[pl.ds(i * 16, 16)] = best
            return carry0

        lax.fori_loop(0, rpw, row_body, 0)
        pltpu.sync_copy(bestv, out_hbm.at[pl.ds(base * 16, rpw * 16)])

    out = knn(d2flat)
    return out.reshape(p_num, 16)


def _knn_fin_body(b_ref, un_ref, out_ref):
    b = b_ref[...]
    s = jnp.sum(jnp.sqrt(jnp.maximum(b, 1e-12)), axis=1, keepdims=True)
    out_ref[...] = s / un_ref[...]


def _knn_finish(best_cat, un_cat):
    n = best_cat.shape[0]
    return pl.pallas_call(
        _knn_fin_body,
        out_shape=jax.ShapeDtypeStruct((n, 1), jnp.float32),
    )(best_cat, un_cat)


def _prep_layer_params(w1, b1, wx, bx, wf, bf, nw1, nb1, nw2, nb2,
                       rw1, rb1, rwf, rbf, rwx, rbx):
    wxb = jnp.zeros((HID, RS, 128), jnp.float32).at[:, :, 0:3].set(
        wx.reshape(HID, RC, 3)[:, :RS, :]).reshape(HID, RS * 128)
    bxb = jnp.zeros((RS, 128), jnp.float32).at[:, 0:3].set(
        bx.reshape(RC, 3)[:RS, :]).reshape(1, RS * 128)
    wf4 = wf[:, :RS * DIM]
    bf4 = bf[:RS * DIM].reshape(1, RS * DIM)
    nw2p = jnp.zeros((HID, 128), jnp.float32).at[:, 0:1].set(nw2)
    nb2p = jnp.zeros((1, 128), jnp.float32).at[0, 0].set(nb2[0])
    rwxp = jnp.zeros((HID, 128), jnp.float32).at[:, 0:3].set(rwx)
    rbxp = jnp.zeros((1, 128), jnp.float32).at[0, 0:3].set(rbx)
    return (w1, b1.reshape(1, HID), wxb, bxb, wf4, bf4,
            nw1, nb1.reshape(1, HID), nw2p, nb2p,
            rw1, rb1.reshape(1, HID), rwf, rbf.reshape(1, DIM), rwxp, rbxp)


def kernel(xyzs, feats, up0_W1, up0_b1, up0_Wx, up0_bx, up0_Wf, up0_bf,
           num0_W1, num0_b1, num0_W2, num0_b2, ref0_W1, ref0_b1, ref0_Wf,
           ref0_bf, ref0_Wx, ref0_bx, up1_W1, up1_b1, up1_Wx, up1_bx,
           up1_Wf, up1_bf, num1_W1, num1_b1, num1_W2, num1_b2, ref1_W1,
           ref1_b1, ref1_Wf, ref1_bf, ref1_Wx, ref1_bx):
    xyz0 = jnp.transpose(xyzs[0])
    xyz0p = jnp.zeros((N0, 128), jnp.float32).at[:, 0:3].set(xyz0)
    f0 = jnp.transpose(feats[0])

    p0 = _prep_layer_params(up0_W1, up0_b1, up0_Wx, up0_bx, up0_Wf, up0_bf,
                            num0_W1, num0_b1, num0_W2, num0_b2, ref0_W1,
                            ref0_b1, ref0_Wf, ref0_bf, ref0_Wx, ref0_bx)
    p1 = _prep_layer_params(up1_W1, up1_b1, up1_Wx, up1_bx, up1_Wf, up1_bf,
                            num1_W1, num1_b1, num1_W2, num1_b2, ref1_W1,
                            ref1_b1, ref1_Wf, ref1_bf, ref1_Wx, ref1_bx)

    xyz1p, f1, un0 = _run_layer(xyz0p, f0, *p0, bn=512)
    xyz2p, f2, un1 = _run_layer(xyz1p, f1, *p1, bn=1024)

    d2_0 = _run_d2(xyz0p, xyz1p, bp=256)
    d2_1 = _run_d2(xyz1p, xyz2p, bp=256)
    best0 = _run_knn_sc(d2_0)
    best1 = _run_knn_sc(d2_1)
    mdis_cat = _knn_finish(jnp.concatenate([best0, best1], axis=0),
                           jnp.concatenate([un0, un1], axis=0))
    mdis0, mdis1 = mdis_cat[:N0], mdis_cat[N0:]

    xyz1_out = jnp.transpose(xyz1p[:, 0:3])[None]
    xyz2_out = jnp.transpose(xyz2p[:, 0:3])[None]
    f_out = jnp.transpose(f2)[None]
    return (xyz1_out, xyz2_out,
            un0.reshape(1, N0), un1.reshape(1, 4 * N0),
            mdis0.reshape(1, N0), mdis1.reshape(1, 4 * N0),
            f_out)


# speedup vs baseline: 1.2465x; 1.2465x over previous
import functools

import jax
import jax.numpy as jnp
from jax import lax
from jax.experimental import pallas as pl
from jax.experimental.pallas import tpu as pltpu
from jax.experimental.pallas import tpu_sc as plsc

B = 1
N0 = 512
DIM = 128
HID = 256
RC = 8
RS = 4
K = 16

_DOT = functools.partial(
    jax.lax.dot_general, precision=jax.lax.Precision.DEFAULT,
    preferred_element_type=jnp.float32)


def _mm(a, b):
    return _DOT(a, b, (((1,), (0,)), ((), ())))


def _layer_body(xyz_ref, f_ref, w1_ref, b1_ref, wx_ref, bx_ref, wf_ref,
                bf_ref, nw1_ref, nb1_ref, nw2_ref, nb2_ref, rw1_ref, rb1_ref,
                rwf_ref, rbf_ref, rwx_ref, rbx_ref,
                xyz_out_ref, f_out_ref, un_out_ref):
    x = xyz_ref[...]
    f = f_ref[...]
    h = jnp.maximum(_mm(f, w1_ref[...]) + b1_ref[...], 0.0)
    co = _mm(h, wx_ref[...]) + bx_ref[...]
    cf = _mm(h, wf_ref[...]) + bf_ref[...]
    hn = jnp.maximum(_mm(f, nw1_ref[...]) + nb1_ref[...], 0.0)
    logit = _mm(hn, nw2_ref[...]) + nb2_ref[...]
    un = 1.0 + (RC - 1.0) * jax.nn.sigmoid(logit[:, 0:1])
    un_out_ref[...] = un
    for r in range(RS):
        xyz_c = x + co[:, r * 128:(r + 1) * 128]
        m_r = jax.nn.sigmoid(un - (r + 1.0))
        f_c = cf[:, r * 128:(r + 1) * 128] * m_r
        h2 = jnp.maximum(_mm(f_c, rw1_ref[...]) + rb1_ref[...], 0.0)
        f_out_ref[:, r, :] = f_c + _mm(h2, rwf_ref[...]) + rbf_ref[...]
        xyz_out_ref[:, r, :] = xyz_c + _mm(h2, rwx_ref[...]) + rbx_ref[...]


def _run_layer(xyzp, f, w1, b1, wxb, bxb, wf4, bf4, nw1, nb1, nw2p, nb2p,
               rw1, rb1, rwf, rbf, rwxp, rbxp, bn):
    n = f.shape[0]
    grid = (n // bn,)
    row = lambda i: (i, 0)
    row3 = lambda i: (i, 0, 0)
    full2 = lambda i: (0, 0)
    wspec = lambda a: pl.BlockSpec(a.shape, full2)
    in_specs = [
        pl.BlockSpec((bn, 128), row), pl.BlockSpec((bn, 128), row),
        wspec(w1), wspec(b1), wspec(wxb), wspec(bxb), wspec(wf4), wspec(bf4),
        wspec(nw1), wspec(nb1), wspec(nw2p), wspec(nb2p),
        wspec(rw1), wspec(rb1), wspec(rwf), wspec(rbf), wspec(rwxp),
        wspec(rbxp),
    ]
    out_shape = [
        jax.ShapeDtypeStruct((n, RS, 128), jnp.float32),
        jax.ShapeDtypeStruct((n, RS, 128), jnp.float32),
        jax.ShapeDtypeStruct((n, 1), jnp.float32),
    ]
    out_specs = [
        pl.BlockSpec((bn, RS, 128), row3),
        pl.BlockSpec((bn, RS, 128), row3),
        pl.BlockSpec((bn, 1), row),
    ]
    xyz_out, f_out, un = pl.pallas_call(
        _layer_body, grid=grid, in_specs=in_specs, out_specs=out_specs,
        out_shape=out_shape)(
            xyzp, f, w1, b1, wxb, bxb, wf4, bf4, nw1, nb1, nw2p, nb2p,
            rw1, rb1, rwf, rbf, rwxp, rbxp)
    return (xyz_out.reshape(n * RS, 128), f_out.reshape(n * RS, 128), un)


def _knn_body(prev_ref, currt_ref, un_ref, mdis_ref):
    pv = prev_ref[...]
    ct = currt_ref[...]
    pn = jnp.sum(pv * pv, axis=1, keepdims=True)
    cn = jnp.sum(ct * ct, axis=0, keepdims=True)
    d2 = pn + cn - 2.0 * _mm(pv, ct)
    big = jnp.float32(3.0e38)
    acc = jnp.zeros_like(pn)
    rem = jnp.full_like(pn, float(K))
    vals = d2
    for _ in range(K):
        m = jnp.min(vals, axis=1, keepdims=True)
        eq = vals == m
        c = jnp.sum(eq.astype(jnp.float32), axis=1, keepdims=True)
        t = jnp.minimum(c, rem)
        acc = acc + jnp.where(
            t > 0.0, jnp.sqrt(jnp.maximum(m, 1e-12)) * t, 0.0)
        rem = rem - t
        vals = jnp.where(eq, big, vals)
    mdis_ref[...] = acc / un_ref[...]


def _run_knn(prevp, currp, un, bp):
    p = prevp.shape[0]
    c = currp.shape[0]
    currt = currp.T
    grid = (p // bp,)
    row = lambda i: (i, 0)
    full2 = lambda i: (0, 0)
    mdis = pl.pallas_call(
        _knn_body, grid=grid,
        in_specs=[pl.BlockSpec((bp, 128), row),
                  pl.BlockSpec((128, c), full2),
                  pl.BlockSpec((bp, 1), row)],
        out_specs=pl.BlockSpec((bp, 1), row),
        out_shape=jax.ShapeDtypeStruct((p, 1), jnp.float32),
    )(prevp, currt, un)
    return mdis


def _d2_body(prev_ref, currt_ref, out_ref):
    pv = prev_ref[...]
    ct = currt_ref[...]
    pn = jnp.sum(pv * pv, axis=1, keepdims=True)
    cn = jnp.sum(ct * ct, axis=0, keepdims=True)
    out_ref[...] = pn + cn - 2.0 * _mm(pv, ct)


def _run_d2(prevp, currp, bp):
    p = prevp.shape[0]
    c = currp.shape[0]
    currt = currp.T
    row = lambda i: (i, 0)
    full2 = lambda i: (0, 0)
    return pl.pallas_call(
        _d2_body, grid=(p // bp,),
        in_specs=[pl.BlockSpec((bp, 128), row),
                  pl.BlockSpec((128, c), full2)],
        out_specs=pl.BlockSpec((bp, c), row),
        out_shape=jax.ShapeDtypeStruct((p, c), jnp.float32),
    )(prevp, currt)


def _sc_iota():
    return lax.iota(jnp.int32, 16)


def _sc_gather_xor(v, j):
    idx = _sc_iota() ^ j
    return lax.gather(
        v, idx[:, None],
        dimension_numbers=lax.GatherDimensionNumbers(
            offset_dims=(), collapsed_slice_dims=(0,), start_index_map=(0,)),
        slice_sizes=(1,), mode=lax.GatherScatterMode.PROMISE_IN_BOUNDS)


def _sc_bfly(v, op):
    for j in (1, 2, 4, 8):
        v = op(v, _sc_gather_xor(v, j))
    return v


def _sc_sort16(v):
    io = _sc_iota()
    for k in (2, 4, 8, 16):
        lk = k.bit_length() - 1
        j = k // 2
        while j >= 1:
            lj = j.bit_length() - 1
            pr = _sc_gather_xor(v, j)
            lo = jnp.minimum(v, pr)
            hi = jnp.maximum(v, pr)
            dirpos = ((io >> lk) ^ (io >> lj)) & 1
            v = jnp.where(dirpos == 0, lo, hi)
            j //= 2
    return v


def _sc_merge16(best, vs):
    io = _sc_iota()
    b2 = jnp.minimum(best, lax.rev(vs, (0,)))
    for j in (8, 4, 2, 1):
        pr = _sc_gather_xor(b2, j)
        lo = jnp.minimum(b2, pr)
        hi = jnp.maximum(b2, pr)
        b2 = jnp.where((io & j) == 0, lo, hi)
    return b2


_SC_BIG = 3.0e38


def _run_knn_sc(d2, rows_hint=None):
    p_num, c_num = d2.shape
    nw = 32
    rpw = p_num // nw
    nsup = c_num // 128
    d2flat = d2.reshape(-1)
    mesh = plsc.VectorSubcoreMesh(core_axis_name="c", subcore_axis_name="s")

    @functools.partial(
        pl.kernel, mesh=mesh,
        out_type=jax.ShapeDtypeStruct((p_num * 16,), jnp.float32),
        scratch_types=[
            pltpu.VMEM((c_num,), jnp.float32),
            pltpu.VMEM((c_num,), jnp.float32),
            pltpu.VMEM((c_num,), jnp.float32),
            pltpu.VMEM((rpw * 16,), jnp.float32),
            pltpu.SMEM((2,), jnp.int32),
        ])
    def knn(d2_hbm, out_hbm, rowv, buf, buf2, bestv, sp):
        wid = lax.axis_index("s") * 2 + lax.axis_index("c")
        base = wid * rpw

        def row_body(i, carry0):
            pltpu.sync_copy(d2_hbm.at[pl.ds((base + i) * c_num, c_num)],
                            rowv)

            def scan_body(s_, carry):
                rm1, rm2 = carry
                off0 = s_ * 128
                d = [rowv[pl.ds(off0 + j * 16, 16)] for j in range(8)]
                m01 = jnp.minimum(d[0], d[1])
                m23 = jnp.minimum(d[2], d[3])
                m45 = jnp.minimum(d[4], d[5])
                m67 = jnp.minimum(d[6], d[7])
                runm = jnp.minimum(jnp.minimum(m01, m23),
                                   jnp.minimum(m45, m67))
                new1 = jnp.minimum(rm1, runm)
                new2 = jnp.minimum(rm2, jnp.maximum(rm1, runm))
                return new1, new2

            big16 = jnp.full((16,), _SC_BIG, jnp.float32)
            rm1, rm2 = lax.fori_loop(0, nsup, scan_body, (big16, big16))
            t_fin = _sc_merge16(_sc_sort16(rm1), _sc_sort16(rm2))[15]
            sp[0] = 0

            def coll_body(s_, carry):
                off0 = s_ * 128
                d = [rowv[pl.ds(off0 + j * 16, 16)] for j in range(8)]
                m01 = jnp.minimum(d[0], d[1])
                m23 = jnp.minimum(d[2], d[3])
                m45 = jnp.minimum(d[4], d[5])
                m67 = jnp.minimum(d[6], d[7])
                runm = jnp.minimum(jnp.minimum(m01, m23),
                                   jnp.minimum(m45, m67))
                smin = _sc_bfly(runm, jnp.minimum)[0]

                @pl.when(smin <= t_fin)
                def _():
                    ptr = sp[0]
                    for j in range(8):
                        buf[pl.ds(ptr + j * 16, 16)] = d[j]
                    sp[0] = ptr + 128

                return carry

            lax.fori_loop(0, nsup, coll_body, 0)
            nb = sp[0]
            sp[1] = 0

            def refil_body(j, carry):
                v = buf[pl.ds(j * 16, 16)]
                mn = _sc_bfly(v, jnp.minimum)[0]

                @pl.when(mn <= t_fin)
                def _():
                    q = sp[1]
                    buf2[pl.ds(q, 16)] = v
                    sp[1] = q + 16

                return carry

            lax.fori_loop(0, nb // 16, refil_body, 0)

            def merge_body(j, best):
                vs = _sc_sort16(buf2[pl.ds(j * 16, 16)])
                return _sc_merge16(best, vs)

            best = lax.fori_loop(0, sp[1] // 16, merge_body,
                                 jnp.full((16,), _SC_BIG, jnp.float32))
            bestv[pl.ds(i * 16, 16)] = best
            return carry0

        lax.fori_loop(0, rpw, row_body, 0)
        pltpu.sync_copy(bestv, out_hbm.at[pl.ds(base * 16, rpw * 16)])

    out = knn(d2flat)
    return out.reshape(p_num, 16)


def _knn_fin_body(b_ref, un_ref, out_ref):
    b = b_ref[...]
    s = jnp.sum(jnp.sqrt(jnp.maximum(b, 1e-12)), axis=1, keepdims=True)
    out_ref[...] = s / un_ref[...]


def _knn_finish(best_cat, un_cat):
    n = best_cat.shape[0]
    return pl.pallas_call(
        _knn_fin_body,
        out_shape=jax.ShapeDtypeStruct((n, 1), jnp.float32),
    )(best_cat, un_cat)


def _prep_layer_params(w1, b1, wx, bx, wf, bf, nw1, nb1, nw2, nb2,
                       rw1, rb1, rwf, rbf, rwx, rbx):
    wxb = jnp.zeros((HID, RS, 128), jnp.float32).at[:, :, 0:3].set(
        wx.reshape(HID, RC, 3)[:, :RS, :]).reshape(HID, RS * 128)
    bxb = jnp.zeros((RS, 128), jnp.float32).at[:, 0:3].set(
        bx.reshape(RC, 3)[:RS, :]).reshape(1, RS * 128)
    wf4 = wf[:, :RS * DIM]
    bf4 = bf[:RS * DIM].reshape(1, RS * DIM)
    nw2p = jnp.zeros((HID, 128), jnp.float32).at[:, 0:1].set(nw2)
    nb2p = jnp.zeros((1, 128), jnp.float32).at[0, 0].set(nb2[0])
    rwxp = jnp.zeros((HID, 128), jnp.float32).at[:, 0:3].set(rwx)
    rbxp = jnp.zeros((1, 128), jnp.float32).at[0, 0:3].set(rbx)
    return (w1, b1.reshape(1, HID), wxb, bxb, wf4, bf4,
            nw1, nb1.reshape(1, HID), nw2p, nb2p,
            rw1, rb1.reshape(1, HID), rwf, rbf.reshape(1, DIM), rwxp, rbxp)


def kernel(xyzs, feats, up0_W1, up0_b1, up0_Wx, up0_bx, up0_Wf, up0_bf,
           num0_W1, num0_b1, num0_W2, num0_b2, ref0_W1, ref0_b1, ref0_Wf,
           ref0_bf, ref0_Wx, ref0_bx, up1_W1, up1_b1, up1_Wx, up1_bx,
           up1_Wf, up1_bf, num1_W1, num1_b1, num1_W2, num1_b2, ref1_W1,
           ref1_b1, ref1_Wf, ref1_bf, ref1_Wx, ref1_bx):
    xyz0 = jnp.transpose(xyzs[0])
    xyz0p = jnp.zeros((N0, 128), jnp.float32).at[:, 0:3].set(xyz0)
    f0 = jnp.transpose(feats[0])

    p0 = _prep_layer_params(up0_W1, up0_b1, up0_Wx, up0_bx, up0_Wf, up0_bf,
                            num0_W1, num0_b1, num0_W2, num0_b2, ref0_W1,
                            ref0_b1, ref0_Wf, ref0_bf, ref0_Wx, ref0_bx)
    p1 = _prep_layer_params(up1_W1, up1_b1, up1_Wx, up1_bx, up1_Wf, up1_bf,
                            num1_W1, num1_b1, num1_W2, num1_b2, ref1_W1,
                            ref1_b1, ref1_Wf, ref1_bf, ref1_Wx, ref1_bx)

    xyz1p, f1, un0 = _run_layer(xyz0p, f0, *p0, bn=512)
    xyz2p, f2, un1 = _run_layer(xyz1p, f1, *p1, bn=1024)

    d2_0 = _run_d2(xyz0p, xyz1p, bp=256)
    d2_1 = _run_d2(xyz1p, xyz2p, bp=256)
    best0 = _run_knn_sc(d2_0)
    best1 = _run_knn_sc(d2_1)
    mdis_cat = _knn_finish(jnp.concatenate([best0, best1], axis=0),
                           jnp.concatenate([un0, un1], axis=0))
    mdis0, mdis1 = mdis_cat[:N0], mdis_cat[N0:]

    xyz1_out = jnp.transpose(xyz1p[:, 0:3])[None]
    xyz2_out = jnp.transpose(xyz2p[:, 0:3])[None]
    f_out = jnp.transpose(f2)[None]
    return (xyz1_out, xyz2_out,
            un0.reshape(1, N0), un1.reshape(1, 4 * N0),
            mdis0.reshape(1, N0), mdis1.reshape(1, 4 * N0),
            f_out)


# speedup vs baseline: 1.4524x; 1.1651x over previous
import functools

import jax
import jax.numpy as jnp
from jax import lax
from jax.experimental import pallas as pl
from jax.experimental.pallas import tpu as pltpu
from jax.experimental.pallas import tpu_sc as plsc

B = 1
N0 = 512
DIM = 128
HID = 256
RC = 8
RS = 4
K = 16

_DOT = functools.partial(
    jax.lax.dot_general, precision=jax.lax.Precision.DEFAULT,
    preferred_element_type=jnp.float32)


def _mm(a, b):
    return _DOT(a, b, (((1,), (0,)), ((), ())))


def _layer_body(xyz_ref, f_ref, w1_ref, b1_ref, wx_ref, bx_ref, wf_ref,
                bf_ref, nw1_ref, nb1_ref, nw2_ref, nb2_ref, rw1_ref, rb1_ref,
                rwf_ref, rbf_ref, rwx_ref, rbx_ref,
                xyz_out_ref, f_out_ref, un_out_ref):
    x = xyz_ref[...]
    f = f_ref[...]
    h = jnp.maximum(_mm(f, w1_ref[...]) + b1_ref[...], 0.0)
    co = _mm(h, wx_ref[...]) + bx_ref[...]
    cf = _mm(h, wf_ref[...]) + bf_ref[...]
    hn = jnp.maximum(_mm(f, nw1_ref[...]) + nb1_ref[...], 0.0)
    logit = _mm(hn, nw2_ref[...]) + nb2_ref[...]
    un = 1.0 + (RC - 1.0) * jax.nn.sigmoid(logit[:, 0:1])
    un_out_ref[...] = un
    for r in range(RS):
        xyz_c = x + co[:, r * 128:(r + 1) * 128]
        m_r = jax.nn.sigmoid(un - (r + 1.0))
        f_c = cf[:, r * 128:(r + 1) * 128] * m_r
        h2 = jnp.maximum(_mm(f_c, rw1_ref[...]) + rb1_ref[...], 0.0)
        f_out_ref[:, r, :] = f_c + _mm(h2, rwf_ref[...]) + rbf_ref[...]
        xyz_out_ref[:, r, :] = xyz_c + _mm(h2, rwx_ref[...]) + rbx_ref[...]


def _run_layer(xyzp, f, w1, b1, wxb, bxb, wf4, bf4, nw1, nb1, nw2p, nb2p,
               rw1, rb1, rwf, rbf, rwxp, rbxp, bn):
    n = f.shape[0]
    grid = (n // bn,)
    row = lambda i: (i, 0)
    row3 = lambda i: (i, 0, 0)
    full2 = lambda i: (0, 0)
    wspec = lambda a: pl.BlockSpec(a.shape, full2)
    in_specs = [
        pl.BlockSpec((bn, 128), row), pl.BlockSpec((bn, 128), row),
        wspec(w1), wspec(b1), wspec(wxb), wspec(bxb), wspec(wf4), wspec(bf4),
        wspec(nw1), wspec(nb1), wspec(nw2p), wspec(nb2p),
        wspec(rw1), wspec(rb1), wspec(rwf), wspec(rbf), wspec(rwxp),
        wspec(rbxp),
    ]
    out_shape = [
        jax.ShapeDtypeStruct((n, RS, 128), jnp.float32),
        jax.ShapeDtypeStruct((n, RS, 128), jnp.float32),
        jax.ShapeDtypeStruct((n, 1), jnp.float32),
    ]
    out_specs = [
        pl.BlockSpec((bn, RS, 128), row3),
        pl.BlockSpec((bn, RS, 128), row3),
        pl.BlockSpec((bn, 1), row),
    ]
    xyz_out, f_out, un = pl.pallas_call(
        _layer_body, grid=grid, in_specs=in_specs, out_specs=out_specs,
        out_shape=out_shape)(
            xyzp, f, w1, b1, wxb, bxb, wf4, bf4, nw1, nb1, nw2p, nb2p,
            rw1, rb1, rwf, rbf, rwxp, rbxp)
    return (xyz_out.reshape(n * RS, 128), f_out.reshape(n * RS, 128), un)


def _knn_body(prev_ref, currt_ref, un_ref, mdis_ref):
    pv = prev_ref[...]
    ct = currt_ref[...]
    pn = jnp.sum(pv * pv, axis=1, keepdims=True)
    cn = jnp.sum(ct * ct, axis=0, keepdims=True)
    d2 = pn + cn - 2.0 * _mm(pv, ct)
    big = jnp.float32(3.0e38)
    acc = jnp.zeros_like(pn)
    rem = jnp.full_like(pn, float(K))
    vals = d2
    for _ in range(K):
        m = jnp.min(vals, axis=1, keepdims=True)
        eq = vals == m
        c = jnp.sum(eq.astype(jnp.float32), axis=1, keepdims=True)
        t = jnp.minimum(c, rem)
        acc = acc + jnp.where(
            t > 0.0, jnp.sqrt(jnp.maximum(m, 1e-12)) * t, 0.0)
        rem = rem - t
        vals = jnp.where(eq, big, vals)
    mdis_ref[...] = acc / un_ref[...]


def _run_knn(prevp, currp, un, bp):
    p = prevp.shape[0]
    c = currp.shape[0]
    currt = currp.T
    grid = (p // bp,)
    row = lambda i: (i, 0)
    full2 = lambda i: (0, 0)
    mdis = pl.pallas_call(
        _knn_body, grid=grid,
        in_specs=[pl.BlockSpec((bp, 128), row),
                  pl.BlockSpec((128, c), full2),
                  pl.BlockSpec((bp, 1), row)],
        out_specs=pl.BlockSpec((bp, 1), row),
        out_shape=jax.ShapeDtypeStruct((p, 1), jnp.float32),
    )(prevp, currt, un)
    return mdis


def _d2_body(prev_ref, currt_ref, out_ref):
    pv = prev_ref[...]
    ct = currt_ref[...]
    pn = jnp.sum(pv * pv, axis=1, keepdims=True)
    cn = jnp.sum(ct * ct, axis=0, keepdims=True)
    out_ref[...] = pn + cn - 2.0 * _mm(pv, ct)


def _run_d2(prevp, currp, bp):
    p = prevp.shape[0]
    c = currp.shape[0]
    currt = currp.T
    row = lambda i: (i, 0)
    full2 = lambda i: (0, 0)
    return pl.pallas_call(
        _d2_body, grid=(p // bp,),
        in_specs=[pl.BlockSpec((bp, 128), row),
                  pl.BlockSpec((128, c), full2)],
        out_specs=pl.BlockSpec((bp, c), row),
        out_shape=jax.ShapeDtypeStruct((p, c), jnp.float32),
    )(prevp, currt)


def _sc_iota():
    return lax.iota(jnp.int32, 16)


def _sc_gather_xor(v, j):
    idx = _sc_iota() ^ j
    return lax.gather(
        v, idx[:, None],
        dimension_numbers=lax.GatherDimensionNumbers(
            offset_dims=(), collapsed_slice_dims=(0,), start_index_map=(0,)),
        slice_sizes=(1,), mode=lax.GatherScatterMode.PROMISE_IN_BOUNDS)


def _sc_bfly(v, op):
    for j in (1, 2, 4, 8):
        v = op(v, _sc_gather_xor(v, j))
    return v


def _sc_sort16(v):
    io = _sc_iota()
    for k in (2, 4, 8, 16):
        lk = k.bit_length() - 1
        j = k // 2
        while j >= 1:
            lj = j.bit_length() - 1
            pr = _sc_gather_xor(v, j)
            lo = jnp.minimum(v, pr)
            hi = jnp.maximum(v, pr)
            dirpos = ((io >> lk) ^ (io >> lj)) & 1
            v = jnp.where(dirpos == 0, lo, hi)
            j //= 2
    return v


def _sc_merge16(best, vs):
    io = _sc_iota()
    b2 = jnp.minimum(best, lax.rev(vs, (0,)))
    for j in (8, 4, 2, 1):
        pr = _sc_gather_xor(b2, j)
        lo = jnp.minimum(b2, pr)
        hi = jnp.maximum(b2, pr)
        b2 = jnp.where((io & j) == 0, lo, hi)
    return b2


_SC_BIG = 3.0e38


def _run_knn_sc(d2, rows_hint=None):
    p_num, c_num = d2.shape
    nw = 32
    rpw = p_num // nw
    nsup = c_num // 128
    d2flat = d2.reshape(-1)
    mesh = plsc.VectorSubcoreMesh(core_axis_name="c", subcore_axis_name="s")

    @functools.partial(
        pl.kernel, mesh=mesh,
        out_type=jax.ShapeDtypeStruct((p_num * 16,), jnp.float32),
        scratch_types=[
            pltpu.VMEM((c_num,), jnp.float32),
            pltpu.VMEM((c_num,), jnp.float32),
            pltpu.VMEM((c_num,), jnp.float32),
            pltpu.VMEM((c_num,), jnp.float32),
            pltpu.VMEM((rpw * 16,), jnp.float32),
            pltpu.SMEM((2,), jnp.int32),
            pltpu.SemaphoreType.DMA,
            pltpu.SemaphoreType.DMA,
        ])
    def knn(d2_hbm, out_hbm, rowv_a, rowv_b, buf, buf2, bestv, sp,
            sem_a, sem_b):
        wid = lax.axis_index("s") * 2 + lax.axis_index("c")
        base = wid * rpw

        def start(buf_ref, sem, i):
            pltpu.async_copy(
                d2_hbm.at[pl.ds((base + i) * c_num, c_num)], buf_ref, sem)

        def drain(buf_ref, sem):
            pltpu.make_async_copy(
                d2_hbm.at[pl.ds(0, c_num)], buf_ref, sem).wait()

        def process(i, rowv):
            def scan_body(s_, carry):
                rm1, rm2 = carry
                off0 = s_ * 128
                d = [rowv[pl.ds(off0 + j * 16, 16)] for j in range(8)]
                m01 = jnp.minimum(d[0], d[1])
                m23 = jnp.minimum(d[2], d[3])
                m45 = jnp.minimum(d[4], d[5])
                m67 = jnp.minimum(d[6], d[7])
                runm = jnp.minimum(jnp.minimum(m01, m23),
                                   jnp.minimum(m45, m67))
                new1 = jnp.minimum(rm1, runm)
                new2 = jnp.minimum(rm2, jnp.maximum(rm1, runm))
                return new1, new2

            big16 = jnp.full((16,), _SC_BIG, jnp.float32)
            rm1, rm2 = lax.fori_loop(0, nsup, scan_body, (big16, big16))
            t_fin = _sc_merge16(_sc_sort16(rm1), _sc_sort16(rm2))[15]
            sp[0] = 0

            def coll_body(s_, carry):
                off0 = s_ * 128
                d = [rowv[pl.ds(off0 + j * 16, 16)] for j in range(8)]
                m01 = jnp.minimum(d[0], d[1])
                m23 = jnp.minimum(d[2], d[3])
                m45 = jnp.minimum(d[4], d[5])
                m67 = jnp.minimum(d[6], d[7])
                runm = jnp.minimum(jnp.minimum(m01, m23),
                                   jnp.minimum(m45, m67))
                smin = _sc_bfly(runm, jnp.minimum)[0]

                @pl.when(smin <= t_fin)
                def _():
                    ptr = sp[0]
                    for j in range(8):
                        buf[pl.ds(ptr + j * 16, 16)] = d[j]
                    sp[0] = ptr + 128

                return carry

            lax.fori_loop(0, nsup, coll_body, 0)
            nb = sp[0]
            sp[1] = 0

            def refil_body(j, carry):
                v = buf[pl.ds(j * 16, 16)]
                mn = _sc_bfly(v, jnp.minimum)[0]

                @pl.when(mn <= t_fin)
                def _():
                    q = sp[1]
                    buf2[pl.ds(q, 16)] = v
                    sp[1] = q + 16

                return carry

            lax.fori_loop(0, nb // 16, refil_body, 0)

            def merge_body(j, best):
                vs = _sc_sort16(buf2[pl.ds(j * 16, 16)])
                return _sc_merge16(best, vs)

            best = lax.fori_loop(0, sp[1] // 16, merge_body,
                                 jnp.full((16,), _SC_BIG, jnp.float32))
            bestv[pl.ds(i * 16, 16)] = best

        start(rowv_a, sem_a, 0)

        def pair_body(g, carry0):
            i0 = 2 * g
            start(rowv_b, sem_b, i0 + 1)
            drain(rowv_a, sem_a)
            process(i0, rowv_a)
            start(rowv_a, sem_a, jnp.minimum(i0 + 2, rpw - 1))
            drain(rowv_b, sem_b)
            process(i0 + 1, rowv_b)
            return carry0

        lax.fori_loop(0, rpw // 2, pair_body, 0)
        drain(rowv_a, sem_a)
        pltpu.sync_copy(bestv, out_hbm.at[pl.ds(base * 16, rpw * 16)])

    out = knn(d2flat)
    return out.reshape(p_num, 16)


def _knn_fin_body(b_ref, un_ref, out_ref):
    b = b_ref[...]
    s = jnp.sum(jnp.sqrt(jnp.maximum(b, 1e-12)), axis=1, keepdims=True)
    out_ref[...] = s / un_ref[...]


def _knn_finish(best_cat, un_cat):
    n = best_cat.shape[0]
    return pl.pallas_call(
        _knn_fin_body,
        out_shape=jax.ShapeDtypeStruct((n, 1), jnp.float32),
    )(best_cat, un_cat)


def _prep_layer_params(w1, b1, wx, bx, wf, bf, nw1, nb1, nw2, nb2,
                       rw1, rb1, rwf, rbf, rwx, rbx):
    wxb = jnp.zeros((HID, RS, 128), jnp.float32).at[:, :, 0:3].set(
        wx.reshape(HID, RC, 3)[:, :RS, :]).reshape(HID, RS * 128)
    bxb = jnp.zeros((RS, 128), jnp.float32).at[:, 0:3].set(
        bx.reshape(RC, 3)[:RS, :]).reshape(1, RS * 128)
    wf4 = wf[:, :RS * DIM]
    bf4 = bf[:RS * DIM].reshape(1, RS * DIM)
    nw2p = jnp.zeros((HID, 128), jnp.float32).at[:, 0:1].set(nw2)
    nb2p = jnp.zeros((1, 128), jnp.float32).at[0, 0].set(nb2[0])
    rwxp = jnp.zeros((HID, 128), jnp.float32).at[:, 0:3].set(rwx)
    rbxp = jnp.zeros((1, 128), jnp.float32).at[0, 0:3].set(rbx)
    return (w1, b1.reshape(1, HID), wxb, bxb, wf4, bf4,
            nw1, nb1.reshape(1, HID), nw2p, nb2p,
            rw1, rb1.reshape(1, HID), rwf, rbf.reshape(1, DIM), rwxp, rbxp)


def kernel(xyzs, feats, up0_W1, up0_b1, up0_Wx, up0_bx, up0_Wf, up0_bf,
           num0_W1, num0_b1, num0_W2, num0_b2, ref0_W1, ref0_b1, ref0_Wf,
           ref0_bf, ref0_Wx, ref0_bx, up1_W1, up1_b1, up1_Wx, up1_bx,
           up1_Wf, up1_bf, num1_W1, num1_b1, num1_W2, num1_b2, ref1_W1,
           ref1_b1, ref1_Wf, ref1_bf, ref1_Wx, ref1_bx):
    xyz0 = jnp.transpose(xyzs[0])
    xyz0p = jnp.zeros((N0, 128), jnp.float32).at[:, 0:3].set(xyz0)
    f0 = jnp.transpose(feats[0])

    p0 = _prep_layer_params(up0_W1, up0_b1, up0_Wx, up0_bx, up0_Wf, up0_bf,
                            num0_W1, num0_b1, num0_W2, num0_b2, ref0_W1,
                            ref0_b1, ref0_Wf, ref0_bf, ref0_Wx, ref0_bx)
    p1 = _prep_layer_params(up1_W1, up1_b1, up1_Wx, up1_bx, up1_Wf, up1_bf,
                            num1_W1, num1_b1, num1_W2, num1_b2, ref1_W1,
                            ref1_b1, ref1_Wf, ref1_bf, ref1_Wx, ref1_bx)

    xyz1p, f1, un0 = _run_layer(xyz0p, f0, *p0, bn=512)
    xyz2p, f2, un1 = _run_layer(xyz1p, f1, *p1, bn=1024)

    d2_0 = _run_d2(xyz0p, xyz1p, bp=256)
    d2_1 = _run_d2(xyz1p, xyz2p, bp=256)
    best0 = _run_knn_sc(d2_0)
    best1 = _run_knn_sc(d2_1)
    mdis_cat = _knn_finish(jnp.concatenate([best0, best1], axis=0),
                           jnp.concatenate([un0, un1], axis=0))
    mdis0, mdis1 = mdis_cat[:N0], mdis_cat[N0:]

    xyz1_out = jnp.transpose(xyz1p[:, 0:3])[None]
    xyz2_out = jnp.transpose(xyz2p[:, 0:3])[None]
    f_out = jnp.transpose(f2)[None]
    return (xyz1_out, xyz2_out,
            un0.reshape(1, N0), un1.reshape(1, 4 * N0),
            mdis0.reshape(1, N0), mdis1.reshape(1, 4 * N0),
            f_out)


# speedup vs baseline: 1.8702x; 1.2877x over previous
import functools

import jax
import jax.numpy as jnp
from jax import lax
from jax.experimental import pallas as pl
from jax.experimental.pallas import tpu as pltpu
from jax.experimental.pallas import tpu_sc as plsc

B = 1
N0 = 512
DIM = 128
HID = 256
RC = 8
RS = 4
K = 16

_DOT = functools.partial(
    jax.lax.dot_general, precision=jax.lax.Precision.DEFAULT,
    preferred_element_type=jnp.float32)


def _mm(a, b):
    return _DOT(a, b, (((1,), (0,)), ((), ())))


def _layer_body(xyz_ref, f_ref, w1_ref, b1_ref, wx_ref, bx_ref, wf_ref,
                bf_ref, nw1_ref, nb1_ref, nw2_ref, nb2_ref, rw1_ref, rb1_ref,
                rwf_ref, rbf_ref, rwx_ref, rbx_ref,
                xyz_out_ref, f_out_ref, un_out_ref):
    x = xyz_ref[...]
    f = f_ref[...]
    h = jnp.maximum(_mm(f, w1_ref[...]) + b1_ref[...], 0.0)
    co = _mm(h, wx_ref[...]) + bx_ref[...]
    cf = _mm(h, wf_ref[...]) + bf_ref[...]
    hn = jnp.maximum(_mm(f, nw1_ref[...]) + nb1_ref[...], 0.0)
    logit = _mm(hn, nw2_ref[...]) + nb2_ref[...]
    un = 1.0 + (RC - 1.0) * jax.nn.sigmoid(logit[:, 0:1])
    un_out_ref[...] = un
    for r in range(RS):
        xyz_c = x + co[:, r * 128:(r + 1) * 128]
        m_r = jax.nn.sigmoid(un - (r + 1.0))
        f_c = cf[:, r * 128:(r + 1) * 128] * m_r
        h2 = jnp.maximum(_mm(f_c, rw1_ref[...]) + rb1_ref[...], 0.0)
        f_out_ref[:, r, :] = f_c + _mm(h2, rwf_ref[...]) + rbf_ref[...]
        xyz_out_ref[:, r, :] = xyz_c + _mm(h2, rwx_ref[...]) + rbx_ref[...]


def _run_layer(xyzp, f, w1, b1, wxb, bxb, wf4, bf4, nw1, nb1, nw2p, nb2p,
               rw1, rb1, rwf, rbf, rwxp, rbxp, bn):
    n = f.shape[0]
    grid = (n // bn,)
    row = lambda i: (i, 0)
    row3 = lambda i: (i, 0, 0)
    full2 = lambda i: (0, 0)
    wspec = lambda a: pl.BlockSpec(a.shape, full2)
    in_specs = [
        pl.BlockSpec((bn, 128), row), pl.BlockSpec((bn, 128), row),
        wspec(w1), wspec(b1), wspec(wxb), wspec(bxb), wspec(wf4), wspec(bf4),
        wspec(nw1), wspec(nb1), wspec(nw2p), wspec(nb2p),
        wspec(rw1), wspec(rb1), wspec(rwf), wspec(rbf), wspec(rwxp),
        wspec(rbxp),
    ]
    out_shape = [
        jax.ShapeDtypeStruct((n, RS, 128), jnp.float32),
        jax.ShapeDtypeStruct((n, RS, 128), jnp.float32),
        jax.ShapeDtypeStruct((n, 1), jnp.float32),
    ]
    out_specs = [
        pl.BlockSpec((bn, RS, 128), row3),
        pl.BlockSpec((bn, RS, 128), row3),
        pl.BlockSpec((bn, 1), row),
    ]
    xyz_out, f_out, un = pl.pallas_call(
        _layer_body, grid=grid, in_specs=in_specs, out_specs=out_specs,
        out_shape=out_shape)(
            xyzp, f, w1, b1, wxb, bxb, wf4, bf4, nw1, nb1, nw2p, nb2p,
            rw1, rb1, rwf, rbf, rwxp, rbxp)
    return (xyz_out.reshape(n * RS, 128), f_out.reshape(n * RS, 128), un)


def _knn_body(prev_ref, currt_ref, un_ref, mdis_ref):
    pv = prev_ref[...]
    ct = currt_ref[...]
    pn = jnp.sum(pv * pv, axis=1, keepdims=True)
    cn = jnp.sum(ct * ct, axis=0, keepdims=True)
    d2 = pn + cn - 2.0 * _mm(pv, ct)
    big = jnp.float32(3.0e38)
    acc = jnp.zeros_like(pn)
    rem = jnp.full_like(pn, float(K))
    vals = d2
    for _ in range(K):
        m = jnp.min(vals, axis=1, keepdims=True)
        eq = vals == m
        c = jnp.sum(eq.astype(jnp.float32), axis=1, keepdims=True)
        t = jnp.minimum(c, rem)
        acc = acc + jnp.where(
            t > 0.0, jnp.sqrt(jnp.maximum(m, 1e-12)) * t, 0.0)
        rem = rem - t
        vals = jnp.where(eq, big, vals)
    mdis_ref[...] = acc / un_ref[...]


def _run_knn(prevp, currp, un, bp):
    p = prevp.shape[0]
    c = currp.shape[0]
    currt = currp.T
    grid = (p // bp,)
    row = lambda i: (i, 0)
    full2 = lambda i: (0, 0)
    mdis = pl.pallas_call(
        _knn_body, grid=grid,
        in_specs=[pl.BlockSpec((bp, 128), row),
                  pl.BlockSpec((128, c), full2),
                  pl.BlockSpec((bp, 1), row)],
        out_specs=pl.BlockSpec((bp, 1), row),
        out_shape=jax.ShapeDtypeStruct((p, 1), jnp.float32),
    )(prevp, currt, un)
    return mdis


def _d2_body(prev_ref, currt_ref, out_ref):
    pv = prev_ref[...]
    ct = currt_ref[...]
    pn = jnp.sum(pv * pv, axis=1, keepdims=True)
    cn = jnp.sum(ct * ct, axis=0, keepdims=True)
    out_ref[...] = pn + cn - 2.0 * _mm(pv, ct)


def _run_d2(prevp, currp, bp):
    p = prevp.shape[0]
    c = currp.shape[0]
    currt = currp.T
    row = lambda i: (i, 0)
    full2 = lambda i: (0, 0)
    return pl.pallas_call(
        _d2_body, grid=(p // bp,),
        in_specs=[pl.BlockSpec((bp, 128), row),
                  pl.BlockSpec((128, c), full2)],
        out_specs=pl.BlockSpec((bp, c), row),
        out_shape=jax.ShapeDtypeStruct((p, c), jnp.float32),
    )(prevp, currt)


def _sc_iota():
    return lax.iota(jnp.int32, 16)


def _sc_gather_xor(v, j):
    idx = _sc_iota() ^ j
    return lax.gather(
        v, idx[:, None],
        dimension_numbers=lax.GatherDimensionNumbers(
            offset_dims=(), collapsed_slice_dims=(0,), start_index_map=(0,)),
        slice_sizes=(1,), mode=lax.GatherScatterMode.PROMISE_IN_BOUNDS)


def _sc_bfly(v, op):
    for j in (1, 2, 4, 8):
        v = op(v, _sc_gather_xor(v, j))
    return v


def _sc_sort16(v):
    io = _sc_iota()
    for k in (2, 4, 8, 16):
        lk = k.bit_length() - 1
        j = k // 2
        while j >= 1:
            lj = j.bit_length() - 1
            pr = _sc_gather_xor(v, j)
            lo = jnp.minimum(v, pr)
            hi = jnp.maximum(v, pr)
            dirpos = ((io >> lk) ^ (io >> lj)) & 1
            v = jnp.where(dirpos == 0, lo, hi)
            j //= 2
    return v


def _sc_merge16(best, vs):
    io = _sc_iota()
    b2 = jnp.minimum(best, lax.rev(vs, (0,)))
    for j in (8, 4, 2, 1):
        pr = _sc_gather_xor(b2, j)
        lo = jnp.minimum(b2, pr)
        hi = jnp.maximum(b2, pr)
        b2 = jnp.where((io & j) == 0, lo, hi)
    return b2


_SC_BIG = 3.0e38


def _run_knn_sc(d2, rows_hint=None):
    p_num, c_num = d2.shape
    nw = 32
    rpw = p_num // nw
    nsup = c_num // 128
    d2flat = d2.reshape(-1)
    mesh = plsc.VectorSubcoreMesh(core_axis_name="c", subcore_axis_name="s")

    @functools.partial(
        pl.kernel, mesh=mesh,
        out_type=jax.ShapeDtypeStruct((p_num * 16,), jnp.float32),
        scratch_types=[
            pltpu.VMEM((c_num,), jnp.float32),
            pltpu.VMEM((c_num,), jnp.float32),
            pltpu.VMEM((c_num,), jnp.float32),
            pltpu.VMEM((c_num,), jnp.float32),
            pltpu.VMEM((rpw * 16,), jnp.float32),
            pltpu.SMEM((2,), jnp.int32),
            pltpu.SemaphoreType.DMA,
            pltpu.SemaphoreType.DMA,
        ])
    def knn(d2_hbm, out_hbm, rowv_a, rowv_b, buf, buf2, bestv, sp,
            sem_a, sem_b):
        wid = lax.axis_index("s") * 2 + lax.axis_index("c")
        base = wid * rpw

        def start(buf_ref, sem, i):
            pltpu.async_copy(
                d2_hbm.at[pl.ds((base + i) * c_num, c_num)], buf_ref, sem)

        def drain(buf_ref, sem):
            pltpu.make_async_copy(
                d2_hbm.at[pl.ds(0, c_num)], buf_ref, sem).wait()

        def process(i, rowv):
            def scan_body(s_, carry):
                rm1, rm2 = carry
                off0 = s_ * 128
                d = [rowv[pl.ds(off0 + j * 16, 16)] for j in range(8)]
                m01 = jnp.minimum(d[0], d[1])
                m23 = jnp.minimum(d[2], d[3])
                m45 = jnp.minimum(d[4], d[5])
                m67 = jnp.minimum(d[6], d[7])
                runm = jnp.minimum(jnp.minimum(m01, m23),
                                   jnp.minimum(m45, m67))
                new1 = jnp.minimum(rm1, runm)
                new2 = jnp.minimum(rm2, jnp.maximum(rm1, runm))
                return new1, new2

            big16 = jnp.full((16,), _SC_BIG, jnp.float32)
            rm1, rm2 = lax.fori_loop(0, nsup, scan_body, (big16, big16))
            t_fin = _sc_merge16(_sc_sort16(rm1), _sc_sort16(rm2))[15]
            sp[0] = 0

            def coll_body(s_, carry):
                off0 = s_ * 128
                d = [rowv[pl.ds(off0 + j * 16, 16)] for j in range(8)]
                m01 = jnp.minimum(d[0], d[1])
                m23 = jnp.minimum(d[2], d[3])
                m45 = jnp.minimum(d[4], d[5])
                m67 = jnp.minimum(d[6], d[7])
                runm = jnp.minimum(jnp.minimum(m01, m23),
                                   jnp.minimum(m45, m67))
                smin = _sc_bfly(runm, jnp.minimum)[0]

                @pl.when(smin <= t_fin)
                def _():
                    ptr = sp[0]
                    for j in range(8):
                        buf[pl.ds(ptr + j * 16, 16)] = d[j]
                    sp[0] = ptr + 128

                return carry

            lax.fori_loop(0, nsup, coll_body, 0)
            nb = sp[0]
            sp[1] = 0

            def refil_body(j, carry):
                v = buf[pl.ds(j * 16, 16)]
                mn = _sc_bfly(v, jnp.minimum)[0]

                @pl.when(mn <= t_fin)
                def _():
                    q = sp[1]
                    buf2[pl.ds(q, 16)] = v
                    sp[1] = q + 16

                return carry

            lax.fori_loop(0, nb // 16, refil_body, 0)

            def merge_body(j, best):
                vs = _sc_sort16(buf2[pl.ds(j * 16, 16)])
                return _sc_merge16(best, vs)

            best = lax.fori_loop(0, sp[1] // 16, merge_body,
                                 jnp.full((16,), _SC_BIG, jnp.float32))
            bestv[pl.ds(i * 16, 16)] = best

        start(rowv_a, sem_a, 0)

        def pair_body(g, carry0):
            i0 = 2 * g
            start(rowv_b, sem_b, i0 + 1)
            drain(rowv_a, sem_a)
            process(i0, rowv_a)
            start(rowv_a, sem_a, jnp.minimum(i0 + 2, rpw - 1))
            drain(rowv_b, sem_b)
            process(i0 + 1, rowv_b)
            return carry0

        lax.fori_loop(0, rpw // 2, pair_body, 0)
        drain(rowv_a, sem_a)
        pltpu.sync_copy(bestv, out_hbm.at[pl.ds(base * 16, rpw * 16)])

    out = knn(d2flat)
    return out.reshape(p_num, 16)


def _knn_fin_body(b_ref, un_ref, out_ref):
    b = b_ref[...]
    s = jnp.sum(jnp.sqrt(jnp.maximum(b, 1e-12)), axis=1, keepdims=True)
    out_ref[...] = s / un_ref[...]


def _knn_finish(best_cat, un_cat):
    n = best_cat.shape[0]
    return pl.pallas_call(
        _knn_fin_body,
        out_shape=jax.ShapeDtypeStruct((n, 1), jnp.float32),
    )(best_cat, un_cat)


def _prep_layer_params(w1, b1, wx, bx, wf, bf, nw1, nb1, nw2, nb2,
                       rw1, rb1, rwf, rbf, rwx, rbx):
    wxb = jnp.zeros((HID, RS, 128), jnp.float32).at[:, :, 0:3].set(
        wx.reshape(HID, RC, 3)[:, :RS, :]).reshape(HID, RS * 128)
    bxb = jnp.zeros((RS, 128), jnp.float32).at[:, 0:3].set(
        bx.reshape(RC, 3)[:RS, :]).reshape(1, RS * 128)
    wf4 = wf[:, :RS * DIM]
    bf4 = bf[:RS * DIM].reshape(1, RS * DIM)
    nw2p = jnp.zeros((HID, 128), jnp.float32).at[:, 0:1].set(nw2)
    nb2p = jnp.zeros((1, 128), jnp.float32).at[0, 0].set(nb2[0])
    rwxp = jnp.zeros((HID, 128), jnp.float32).at[:, 0:3].set(rwx)
    rbxp = jnp.zeros((1, 128), jnp.float32).at[0, 0:3].set(rbx)
    return (w1, b1.reshape(1, HID), wxb, bxb, wf4, bf4,
            nw1, nb1.reshape(1, HID), nw2p, nb2p,
            rw1, rb1.reshape(1, HID), rwf, rbf.reshape(1, DIM), rwxp, rbxp)


def kernel(xyzs, feats, up0_W1, up0_b1, up0_Wx, up0_bx, up0_Wf, up0_bf,
           num0_W1, num0_b1, num0_W2, num0_b2, ref0_W1, ref0_b1, ref0_Wf,
           ref0_bf, ref0_Wx, ref0_bx, up1_W1, up1_b1, up1_Wx, up1_bx,
           up1_Wf, up1_bf, num1_W1, num1_b1, num1_W2, num1_b2, ref1_W1,
           ref1_b1, ref1_Wf, ref1_bf, ref1_Wx, ref1_bx):
    xyz0 = jnp.transpose(xyzs[0])
    xyz0p = jnp.zeros((N0, 128), jnp.float32).at[:, 0:3].set(xyz0)
    f0 = jnp.transpose(feats[0])

    p0 = _prep_layer_params(up0_W1, up0_b1, up0_Wx, up0_bx, up0_Wf, up0_bf,
                            num0_W1, num0_b1, num0_W2, num0_b2, ref0_W1,
                            ref0_b1, ref0_Wf, ref0_bf, ref0_Wx, ref0_bx)
    p1 = _prep_layer_params(up1_W1, up1_b1, up1_Wx, up1_bx, up1_Wf, up1_bf,
                            num1_W1, num1_b1, num1_W2, num1_b2, ref1_W1,
                            ref1_b1, ref1_Wf, ref1_bf, ref1_Wx, ref1_bx)

    xyz1p, f1, un0 = _run_layer(xyz0p, f0, *p0, bn=512)
    xyz2p, f2, un1 = _run_layer(xyz1p, f1, *p1, bn=1024)

    rt = 1024
    d2_0 = _run_d2(xyz0p, xyz1p, bp=256)
    d2_1 = _run_d2(xyz1p[rt:], xyz2p, bp=256)
    mdis1_tc = _run_knn(xyz1p[:rt], xyz2p, un1[:rt], bp=512)
    best0 = _run_knn_sc(d2_0)
    best1 = _run_knn_sc(d2_1)
    mdis_cat = _knn_finish(jnp.concatenate([best0, best1], axis=0),
                           jnp.concatenate([un0, un1[rt:]], axis=0))
    mdis0 = mdis_cat[:N0]
    mdis1 = jnp.concatenate([mdis1_tc, mdis_cat[N0:]], axis=0)

    xyz1_out = jnp.transpose(xyz1p[:, 0:3])[None]
    xyz2_out = jnp.transpose(xyz2p[:, 0:3])[None]
    f_out = jnp.transpose(f2)[None]
    return (xyz1_out, xyz2_out,
            un0.reshape(1, N0), un1.reshape(1, 4 * N0),
            mdis0.reshape(1, N0), mdis1.reshape(1, 4 * N0),
            f_out)
